# Initial kernel scaffold; baseline (speedup 1.0000x reference)
import jax
import jax.numpy as jnp
from jax.experimental import pallas as pl


def kernel(x, adj, Wp, bp, Wl, Wr, att, bias, Wh, bh):
    raise NotImplementedError("write your pallas kernel here")


# speedup vs baseline: 2.3604x; 2.3604x over previous
import functools

import jax
import jax.numpy as jnp
from jax import lax
from jax.experimental import pallas as pl
from jax.experimental.pallas import tpu as pltpu
from jax.experimental.pallas import tpu_sc as plsc

NEG_SLOPE = 0.2
NC = 2
NS = 16
LANES = 16
CHUNK = 80
ROW_BLK = 128


def _proj_body(x_ref, wp_ref, bp_ref, wl_ref, wr_ref, h_ref, hl_ref, hr_ref):
    h = jnp.dot(x_ref[...], wp_ref[...],
                preferred_element_type=jnp.float32) + bp_ref[...]
    h_ref[...] = h
    hl_ref[...] = jnp.dot(h, wl_ref[...], preferred_element_type=jnp.float32)
    hr_ref[...] = jnp.dot(h, wr_ref[...], preferred_element_type=jnp.float32)


def _proj(x, wp, bp2, wl, wr):
    n, d = x.shape
    blk = 1000
    grid = n // blk
    row_spec = pl.BlockSpec((blk, d), lambda i: (i, 0))
    w_spec = pl.BlockSpec((d, d), lambda i: (0, 0))
    b_spec = pl.BlockSpec((1, d), lambda i: (0, 0))
    out_sd = jax.ShapeDtypeStruct((n, d), jnp.float32)
    return pl.pallas_call(
        _proj_body,
        grid=(grid,),
        in_specs=[row_spec, w_spec, b_spec, w_spec, w_spec],
        out_specs=[row_spec, row_spec, row_spec],
        out_shape=[out_sd, out_sd, out_sd],
    )(x, wp, bp2, wl, wr)


def _sc_edge_body(hl_hbm, hr_hbm, src_hbm, dst_hbm, attb_hbm,
                  acc_out, den_out,
                  src_c, dst_v, hlrows, hrrows, eexp_v, attb_v,
                  acc_sh, den_sh, sem1, sem2):
    rows_per_tile = acc_sh.shape[0] // NS
    n_chunks = dst_v.shape[0]

    c = lax.axis_index("c")
    s = lax.axis_index("s")
    wid = c * NS + s

    pltpu.sync_copy(dst_hbm.at[wid], dst_v)
    pltpu.sync_copy(attb_hbm, attb_v)

    zeros = jnp.zeros((LANES,), jnp.float32)

    def _z_rows(i, _):
        for g in range(8):
            hlrows[i, pl.ds(g * LANES, LANES)] = zeros
        return 0
    lax.fori_loop(0, CHUNK, _z_rows, 0)

    def _z_ee(i, _):
        eexp_v[i] = zeros
        return 0
    lax.fori_loop(0, CHUNK, _z_ee, 0)

    for j in range(rows_per_tile // CHUNK):
        base = s * rows_per_tile + j * CHUNK
        pltpu.sync_copy(hlrows, acc_sh.at[pl.ds(base, CHUNK)])
        pltpu.sync_copy(eexp_v, den_sh.at[pl.ds(base, CHUNK)])
    plsc.subcore_barrier()

    def _chunk(it, _):
        pltpu.sync_copy(src_hbm.at[wid, it], src_c)
        cp1 = pltpu.async_copy(hl_hbm.at[src_c], hlrows, sem1)
        cp2 = pltpu.async_copy(hr_hbm.at[dst_v.at[it]], hrrows, sem2)
        cp1.wait()
        cp2.wait()
        for g in range(CHUNK // LANES):
            eids = g * LANES + lax.iota(jnp.int32, LANES)

            def _score(j, acc):
                for dd in range(8):
                    dv = j * 8 + dd
                    dspl = jnp.full((LANES,), dv, jnp.int32)
                    a = plsc.load_gather(hlrows, [eids, dspl])
                    b = plsc.load_gather(hrrows, [eids, dspl])
                    z = a + b
                    lr = jnp.maximum(z, NEG_SLOPE * z)
                    acc = acc + lr * attb_v[dv]
                return acc

            acc = lax.fori_loop(0, 16, _score, zeros)
            ee = jnp.exp(acc)
            plsc.store_scatter(eexp_v, [eids, jnp.zeros((LANES,), jnp.int32)],
                               ee)

            def _scale(j, _):
                for dd in range(8):
                    dv = j * 8 + dd
                    dspl = jnp.full((LANES,), dv, jnp.int32)
                    v = plsc.load_gather(hlrows, [eids, dspl])
                    plsc.store_scatter(hlrows, [eids, dspl], v * ee)
                return 0

            lax.fori_loop(0, 16, _scale, 0)
        pltpu.sync_copy(hlrows, acc_sh.at[dst_v.at[it]], add=True)
        pltpu.sync_copy(eexp_v, den_sh.at[dst_v.at[it]], add=True)
        return 0

    lax.fori_loop(0, n_chunks, _chunk, 0)
    plsc.subcore_barrier()

    for j in range(rows_per_tile // CHUNK):
        base = s * rows_per_tile + j * CHUNK
        pltpu.sync_copy(acc_sh.at[pl.ds(base, CHUNK)], hlrows)
        pltpu.sync_copy(hlrows, acc_out.at[c, pl.ds(base, CHUNK)])
        pltpu.sync_copy(den_sh.at[pl.ds(base, CHUNK)], eexp_v)
        pltpu.sync_copy(eexp_v, den_out.at[c, pl.ds(base, CHUNK)])


def _sc_edges(hl, hr, src3d, dst3d, attb, npad):
    n, d = hl.shape
    chunks_per_tile = src3d.shape[1]
    rows_per_tile = npad // NS
    mesh = plsc.VectorSubcoreMesh(core_axis_name="c", subcore_axis_name="s",
                                  num_cores=NC, num_subcores=NS)
    f = pl.kernel(
        _sc_edge_body,
        out_type=[jax.ShapeDtypeStruct((NC, npad, d), jnp.float32),
                  jax.ShapeDtypeStruct((NC, npad, LANES), jnp.float32)],
        mesh=mesh,
        compiler_params=pltpu.CompilerParams(
            needs_layout_passes=False, use_tc_tiling_on_sc=False),
        scratch_types=[
            pltpu.VMEM((CHUNK,), jnp.int32),
            pltpu.VMEM((chunks_per_tile, CHUNK), jnp.int32),
            pltpu.VMEM((CHUNK, d), jnp.float32),
            pltpu.VMEM((CHUNK, d), jnp.float32),
            pltpu.VMEM((CHUNK, LANES), jnp.float32),
            pltpu.VMEM((d, LANES), jnp.float32),
            pltpu.VMEM_SHARED((npad, d), jnp.float32),
            pltpu.VMEM_SHARED((npad, LANES), jnp.float32),
            pltpu.SemaphoreType.DMA,
            pltpu.SemaphoreType.DMA,
        ],
    )
    return f(hl, hr, src3d, dst3d, attb)


def _out_body(a_ref, d_ref, h_ref, bias_ref, wh_ref, bh_ref, o_ref):
    acc = a_ref[0] + a_ref[1]
    den = d_ref[0] + d_ref[1]
    denc = den[:, 0:1]
    o = acc / (denc + 1e-16) + h_ref[...] + bias_ref[...]
    o_ref[...] = jnp.dot(o, wh_ref[...],
                         preferred_element_type=jnp.float32) + bh_ref[...]


def _combine(accs, dens, h, bias2, whp, bhp2):
    n, d = h.shape
    blk = 1000
    grid = n // blk
    row_spec = pl.BlockSpec((blk, d), lambda i: (i, 0))
    acc_spec = pl.BlockSpec((NC, blk, d), lambda i: (0, i, 0))
    den_spec = pl.BlockSpec((NC, blk, LANES), lambda i: (0, i, 0))
    w_spec = pl.BlockSpec((d, d), lambda i: (0, 0))
    b_spec = pl.BlockSpec((1, d), lambda i: (0, 0))
    return pl.pallas_call(
        _out_body,
        grid=(grid,),
        in_specs=[acc_spec, den_spec, row_spec, b_spec, w_spec, b_spec],
        out_specs=row_spec,
        out_shape=jax.ShapeDtypeStruct((n, d), jnp.float32),
    )(accs, dens, h, bias2, whp, bhp2)


def kernel(x, adj, Wp, bp, Wl, Wr, att, bias, Wh, bh):
    n, d = x.shape
    e = adj.shape[1]
    nw = NC * NS
    npad = ((n + NS * ROW_BLK - 1) // (NS * ROW_BLK)) * (NS * ROW_BLK)
    src3d = adj[0].reshape(nw, e // (nw * CHUNK), CHUNK)
    dst3d = adj[1].reshape(nw, e // (nw * CHUNK), CHUNK)
    attb = jnp.broadcast_to(att[:, None], (d, LANES))

    h, hl, hr = _proj(x, Wp, bp.reshape(1, d), Wl, Wr)
    accs, dens = _sc_edges(hl, hr, src3d, dst3d, attb, npad)

    whp = jnp.pad(Wh, ((0, 0), (0, d - Wh.shape[1])))
    bhp2 = jnp.pad(bh, (0, d - bh.shape[0])).reshape(1, d)
    out = _combine(accs, dens, h, bias.reshape(1, d), whp, bhp2)
    return out[:, :1]


# speedup vs baseline: 10.2243x; 4.3316x over previous
import functools

import jax
import jax.numpy as jnp
from jax import lax
from jax.experimental import pallas as pl
from jax.experimental.pallas import tpu as pltpu
from jax.experimental.pallas import tpu_sc as plsc

NEG_SLOPE = 0.2
NC = 2
NS = 16
LANES = 16
CHUNK = 80
EDGE_UNROLL = 4
ROW_BLK = 128


def _proj_body(x_ref, wp_ref, bp_ref, wl_ref, wr_ref, h_ref, hl_ref, hr_ref):
    h = jnp.dot(x_ref[...], wp_ref[...],
                preferred_element_type=jnp.float32) + bp_ref[...]
    h_ref[...] = h
    hl_ref[...] = jnp.dot(h, wl_ref[...], preferred_element_type=jnp.float32)
    hr_ref[...] = jnp.dot(h, wr_ref[...], preferred_element_type=jnp.float32)


def _proj(x, wp, bp2, wl, wr):
    n, d = x.shape
    blk = 1000
    grid = n // blk
    row_spec = pl.BlockSpec((blk, d), lambda i: (i, 0))
    w_spec = pl.BlockSpec((d, d), lambda i: (0, 0))
    b_spec = pl.BlockSpec((1, d), lambda i: (0, 0))
    out_sd = jax.ShapeDtypeStruct((n, d), jnp.float32)
    return pl.pallas_call(
        _proj_body,
        grid=(grid,),
        in_specs=[row_spec, w_spec, b_spec, w_spec, w_spec],
        out_specs=[row_spec, row_spec, row_spec],
        out_shape=[out_sd, out_sd, out_sd],
    )(x, wp, bp2, wl, wr)


def _sc_edge_body(hl_hbm, hr_hbm, src_hbm, dst_hbm, attb_hbm,
                  acc_out, den_out,
                  src_c, dst_v, hlrows, hrrows, eexp_v, attb_v,
                  acc_sh, den_sh, sem1, sem2):
    rows_per_tile = acc_sh.shape[0] // NS
    n_chunks = dst_v.shape[0]

    c = lax.axis_index("c")
    s = lax.axis_index("s")
    wid = c * NS + s

    pltpu.sync_copy(dst_hbm.at[wid], dst_v)
    pltpu.sync_copy(attb_hbm, attb_v)

    zeros = jnp.zeros((LANES,), jnp.float32)

    def _z_rows(i, _):
        for g in range(8):
            hlrows[i, pl.ds(g * LANES, LANES)] = zeros
        return 0
    lax.fori_loop(0, CHUNK, _z_rows, 0)

    def _z_ee(i, _):
        eexp_v[i] = zeros
        return 0
    lax.fori_loop(0, CHUNK, _z_ee, 0)

    for j in range(rows_per_tile // CHUNK):
        base = s * rows_per_tile + j * CHUNK
        pltpu.sync_copy(hlrows, acc_sh.at[pl.ds(base, CHUNK)])
        pltpu.sync_copy(eexp_v, den_sh.at[pl.ds(base, CHUNK)])
    plsc.subcore_barrier()

    av = [attb_v[g] for g in range(8)]
    zcol = jnp.zeros((LANES,), jnp.int32)

    def _chunk(it, _):
        pltpu.sync_copy(src_hbm.at[wid, it], src_c)
        cp1 = pltpu.async_copy(hl_hbm.at[src_c], hlrows, sem1)
        cp2 = pltpu.async_copy(hr_hbm.at[dst_v.at[it]], hrrows, sem2)
        cp1.wait()
        cp2.wait()

        def _edge(e4, _):
            for k in range(EDGE_UNROLL):
                e = e4 * EDGE_UNROLL + k
                acc = zeros
                for g in range(8):
                    a = hlrows[e, pl.ds(g * LANES, LANES)]
                    b = hrrows[e, pl.ds(g * LANES, LANES)]
                    z = a + b
                    acc = acc + jnp.maximum(z, NEG_SLOPE * z) * av[g]
                ee = jnp.exp(jnp.full((LANES,), jnp.sum(acc), jnp.float32))
                for g in range(8):
                    hlrows[e, pl.ds(g * LANES, LANES)] = (
                        hlrows[e, pl.ds(g * LANES, LANES)] * ee)
                plsc.store_scatter(
                    eexp_v, [jnp.full((LANES,), e, jnp.int32), zcol], ee)
            return 0

        lax.fori_loop(0, CHUNK // EDGE_UNROLL, _edge, 0)
        pltpu.sync_copy(hlrows, acc_sh.at[dst_v.at[it]], add=True)
        pltpu.sync_copy(eexp_v, den_sh.at[dst_v.at[it]], add=True)
        return 0

    lax.fori_loop(0, n_chunks, _chunk, 0)
    plsc.subcore_barrier()

    for j in range(rows_per_tile // CHUNK):
        base = s * rows_per_tile + j * CHUNK
        pltpu.sync_copy(acc_sh.at[pl.ds(base, CHUNK)], hlrows)
        pltpu.sync_copy(hlrows, acc_out.at[c, pl.ds(base, CHUNK)])
        pltpu.sync_copy(den_sh.at[pl.ds(base, CHUNK)], eexp_v)
        pltpu.sync_copy(eexp_v, den_out.at[c, pl.ds(base, CHUNK)])


def _sc_edges(hl, hr, src3d, dst3d, attb, npad):
    n, d = hl.shape
    chunks_per_tile = src3d.shape[1]
    rows_per_tile = npad // NS
    mesh = plsc.VectorSubcoreMesh(core_axis_name="c", subcore_axis_name="s",
                                  num_cores=NC, num_subcores=NS)
    f = pl.kernel(
        _sc_edge_body,
        out_type=[jax.ShapeDtypeStruct((NC, npad, d), jnp.float32),
                  jax.ShapeDtypeStruct((NC, npad, LANES), jnp.float32)],
        mesh=mesh,
        compiler_params=pltpu.CompilerParams(
            needs_layout_passes=False, use_tc_tiling_on_sc=False),
        scratch_types=[
            pltpu.VMEM((CHUNK,), jnp.int32),
            pltpu.VMEM((chunks_per_tile, CHUNK), jnp.int32),
            pltpu.VMEM((CHUNK, d), jnp.float32),
            pltpu.VMEM((CHUNK, d), jnp.float32),
            pltpu.VMEM((CHUNK, LANES), jnp.float32),
            pltpu.VMEM((d // LANES, LANES), jnp.float32),
            pltpu.VMEM_SHARED((npad, d), jnp.float32),
            pltpu.VMEM_SHARED((npad, LANES), jnp.float32),
            pltpu.SemaphoreType.DMA,
            pltpu.SemaphoreType.DMA,
        ],
    )
    return f(hl, hr, src3d, dst3d, attb)


def _out_body(a_ref, d_ref, h_ref, bias_ref, wh_ref, bh_ref, o_ref):
    acc = a_ref[0] + a_ref[1]
    den = d_ref[0] + d_ref[1]
    denc = den[:, 0:1]
    o = acc / (denc + 1e-16) + h_ref[...] + bias_ref[...]
    o_ref[...] = jnp.dot(o, wh_ref[...],
                         preferred_element_type=jnp.float32) + bh_ref[...]


def _combine(accs, dens, h, bias2, whp, bhp2):
    n, d = h.shape
    blk = 1000
    grid = n // blk
    row_spec = pl.BlockSpec((blk, d), lambda i: (i, 0))
    acc_spec = pl.BlockSpec((NC, blk, d), lambda i: (0, i, 0))
    den_spec = pl.BlockSpec((NC, blk, LANES), lambda i: (0, i, 0))
    w_spec = pl.BlockSpec((d, d), lambda i: (0, 0))
    b_spec = pl.BlockSpec((1, d), lambda i: (0, 0))
    return pl.pallas_call(
        _out_body,
        grid=(grid,),
        in_specs=[acc_spec, den_spec, row_spec, b_spec, w_spec, b_spec],
        out_specs=row_spec,
        out_shape=jax.ShapeDtypeStruct((n, d), jnp.float32),
    )(accs, dens, h, bias2, whp, bhp2)


def kernel(x, adj, Wp, bp, Wl, Wr, att, bias, Wh, bh):
    n, d = x.shape
    e = adj.shape[1]
    nw = NC * NS
    npad = ((n + NS * ROW_BLK - 1) // (NS * ROW_BLK)) * (NS * ROW_BLK)
    src3d = adj[0].reshape(nw, e // (nw * CHUNK), CHUNK)
    dst3d = adj[1].reshape(nw, e // (nw * CHUNK), CHUNK)
    attb = att.reshape(d // LANES, LANES)

    h, hl, hr = _proj(x, Wp, bp.reshape(1, d), Wl, Wr)
    accs, dens = _sc_edges(hl, hr, src3d, dst3d, attb, npad)

    whp = jnp.pad(Wh, ((0, 0), (0, d - Wh.shape[1])))
    bhp2 = jnp.pad(bh, (0, d - bh.shape[0])).reshape(1, d)
    out = _combine(accs, dens, h, bias.reshape(1, d), whp, bhp2)
    return out[:, :1]


# speedup vs baseline: 14.0310x; 1.3723x over previous
import functools

import jax
import jax.numpy as jnp
from jax import lax
from jax.experimental import pallas as pl
from jax.experimental.pallas import tpu as pltpu
from jax.experimental.pallas import tpu_sc as plsc

NEG_SLOPE = 0.2
NC = 2
NS = 16
LANES = 16
CHUNK = 40
DEPTH = 3
IDEPTH = 6
EDGE_UNROLL = 4


def _proj_body(x_ref, wp_ref, bp_ref, wl_ref, wr_ref, h_ref, hl_ref, hr_ref):
    h = jnp.dot(x_ref[...], wp_ref[...],
                preferred_element_type=jnp.float32) + bp_ref[...]
    h_ref[...] = h
    hl_ref[...] = jnp.dot(h, wl_ref[...], preferred_element_type=jnp.float32)
    hr_ref[...] = jnp.dot(h, wr_ref[...], preferred_element_type=jnp.float32)


def _proj(x, wp, bp2, wl, wr):
    n, d = x.shape
    blk = 1000
    grid = n // blk
    row_spec = pl.BlockSpec((blk, d), lambda i: (i, 0))
    w_spec = pl.BlockSpec((d, d), lambda i: (0, 0))
    b_spec = pl.BlockSpec((1, d), lambda i: (0, 0))
    out_sd = jax.ShapeDtypeStruct((n, d), jnp.float32)
    return pl.pallas_call(
        _proj_body,
        grid=(grid,),
        in_specs=[row_spec, w_spec, b_spec, w_spec, w_spec],
        out_specs=[row_spec, row_spec, row_spec],
        out_shape=[out_sd, out_sd, out_sd],
    )(x, wp, bp2, wl, wr)


def _sc_edge_body(hl_hbm, hr_hbm, src_hbm, dst_hbm, attb_hbm,
                  acc_out, den_out,
                  src_c, dst_c, hlr, hrr, eev, attb_v,
                  acc_sh, den_sh, sin, sout, sidx):
    rows_per_tile = acc_sh.shape[0] // NS
    n_chunks = src_hbm.shape[1]

    c = lax.axis_index("c")
    s = lax.axis_index("s")
    wid = c * NS + s

    pltpu.sync_copy(attb_hbm, attb_v)

    zeros = jnp.zeros((LANES,), jnp.float32)

    def _z_rows(i, _):
        for g in range(8):
            hlr[0][i, pl.ds(g * LANES, LANES)] = zeros
        return 0
    lax.fori_loop(0, CHUNK, _z_rows, 0)

    for b in range(DEPTH):
        def _z_ee(i, _):
            eev[b][i] = zeros
            return 0
        lax.fori_loop(0, CHUNK, _z_ee, 0)

    for j in range(rows_per_tile // CHUNK):
        base = s * rows_per_tile + j * CHUNK
        pltpu.sync_copy(hlr[0], acc_sh.at[pl.ds(base, CHUNK)])
        pltpu.sync_copy(eev[0], den_sh.at[pl.ds(base, CHUNK)])
    plsc.subcore_barrier()

    av = [attb_v[g] for g in range(8)]
    zcol = jnp.zeros((LANES,), jnp.int32)

    def _issue_idx(cc, q):
        pltpu.async_copy(src_hbm.at[wid, cc], src_c[q], sidx[q])
        pltpu.async_copy(dst_hbm.at[wid, cc], dst_c[q], sidx[q])

    def _drain_idx(q):
        pltpu.make_async_copy(src_hbm.at[wid, 0], src_c[q], sidx[q]).wait()
        pltpu.make_async_copy(src_hbm.at[wid, 0], dst_c[q], sidx[q]).wait()

    def _issue_gather(q, b):
        pltpu.async_copy(hl_hbm.at[src_c[q]], hlr[b], sin[b])
        pltpu.async_copy(hr_hbm.at[dst_c[q]], hrr[b], sin[b])

    def _drain_gather(b):
        pltpu.make_async_copy(hl_hbm.at[pl.ds(0, CHUNK)], hlr[b],
                              sin[b]).wait()
        pltpu.make_async_copy(hl_hbm.at[pl.ds(0, CHUNK)], hrr[b],
                              sin[b]).wait()

    def _issue_scatter(q, b):
        pltpu.async_copy(hlr[b], acc_sh.at[dst_c[q]], sout[b], add=True)
        pltpu.async_copy(eev[b], den_sh.at[dst_c[q]], sout[b], add=True)

    def _drain_scatter(q):
        pltpu.make_async_copy(hl_hbm.at[pl.ds(0, CHUNK)], hlr[q],
                              sout[q]).wait()
        pltpu.make_async_copy(den_out.at[0, pl.ds(0, CHUNK)], eev[q],
                              sout[q]).wait()

    def _compute(q):
        def _edge(e4, _):
            for k in range(EDGE_UNROLL):
                e = e4 * EDGE_UNROLL + k
                acc = zeros
                for g in range(8):
                    a = hlr[q][e, pl.ds(g * LANES, LANES)]
                    b = hrr[q][e, pl.ds(g * LANES, LANES)]
                    z = a + b
                    acc = acc + jnp.maximum(z, NEG_SLOPE * z) * av[g]
                ee = jnp.exp(jnp.full((LANES,), jnp.sum(acc), jnp.float32))
                for g in range(8):
                    hlr[q][e, pl.ds(g * LANES, LANES)] = (
                        hlr[q][e, pl.ds(g * LANES, LANES)] * ee)
                plsc.store_scatter(
                    eev[q], [jnp.full((LANES,), e, jnp.int32), zcol], ee)
            return 0

        lax.fori_loop(0, CHUNK // EDGE_UNROLL, _edge, 0)

    _issue_idx(0, 0)
    _issue_idx(1, 1)
    _drain_idx(0)
    _issue_gather(0, 0)

    def _round(i, _):
        for b6 in range(IDEPTH):
            cc = i * IDEPTH + b6
            rb = b6 % DEPTH
            nrb = (rb + 1) % DEPTH

            @pl.when(cc >= 2)
            def _():
                _drain_scatter(nrb)

            @pl.when(cc + 1 < n_chunks)
            def _():
                _drain_idx((b6 + 1) % IDEPTH)
                _issue_gather((b6 + 1) % IDEPTH, nrb)

            @pl.when(cc + 2 < n_chunks)
            def _():
                _issue_idx(cc + 2, (b6 + 2) % IDEPTH)

            _drain_gather(rb)
            _compute(rb)
            _issue_scatter(b6, rb)
        return 0

    lax.fori_loop(0, n_chunks // IDEPTH, _round, 0)
    _drain_scatter((n_chunks - 2) % DEPTH)
    _drain_scatter((n_chunks - 1) % DEPTH)
    plsc.subcore_barrier()

    for j in range(rows_per_tile // CHUNK):
        base = s * rows_per_tile + j * CHUNK
        pltpu.sync_copy(acc_sh.at[pl.ds(base, CHUNK)], hlr[0])
        pltpu.sync_copy(hlr[0], acc_out.at[c, pl.ds(base, CHUNK)])
        pltpu.sync_copy(den_sh.at[pl.ds(base, CHUNK)], eev[0])
        pltpu.sync_copy(eev[0], den_out.at[c, pl.ds(base, CHUNK)])


def _sc_edges(hl, hr, src3d, dst3d, attb, npad):
    n, d = hl.shape
    rows_per_tile = npad // NS
    mesh = plsc.VectorSubcoreMesh(core_axis_name="c", subcore_axis_name="s",
                                  num_cores=NC, num_subcores=NS)
    f = pl.kernel(
        _sc_edge_body,
        out_type=[jax.ShapeDtypeStruct((NC, npad, d), jnp.float32),
                  jax.ShapeDtypeStruct((NC, npad, LANES), jnp.float32)],
        mesh=mesh,
        compiler_params=pltpu.CompilerParams(
            needs_layout_passes=False, use_tc_tiling_on_sc=False),
        scratch_types=[
            [pltpu.VMEM((CHUNK,), jnp.int32)] * IDEPTH,
            [pltpu.VMEM((CHUNK,), jnp.int32)] * IDEPTH,
            [pltpu.VMEM((CHUNK, d), jnp.float32)] * DEPTH,
            [pltpu.VMEM((CHUNK, d), jnp.float32)] * DEPTH,
            [pltpu.VMEM((CHUNK, LANES), jnp.float32)] * DEPTH,
            pltpu.VMEM((d // LANES, LANES), jnp.float32),
            pltpu.VMEM_SHARED((npad, d), jnp.float32),
            pltpu.VMEM_SHARED((npad, LANES), jnp.float32),
            [pltpu.SemaphoreType.DMA] * DEPTH,
            [pltpu.SemaphoreType.DMA] * DEPTH,
            [pltpu.SemaphoreType.DMA] * IDEPTH,
        ],
    )
    return f(hl, hr, src3d, dst3d, attb)


def _out_body(a_ref, d_ref, h_ref, bias_ref, wh_ref, bh_ref, o_ref):
    acc = a_ref[0] + a_ref[1]
    den = d_ref[0] + d_ref[1]
    denc = den[:, 0:1]
    o = acc / (denc + 1e-16) + h_ref[...] + bias_ref[...]
    o_ref[...] = jnp.dot(o, wh_ref[...],
                         preferred_element_type=jnp.float32) + bh_ref[...]


def _combine(accs, dens, h, bias2, whp, bhp2):
    n, d = h.shape
    blk = 1000
    grid = n // blk
    row_spec = pl.BlockSpec((blk, d), lambda i: (i, 0))
    acc_spec = pl.BlockSpec((NC, blk, d), lambda i: (0, i, 0))
    den_spec = pl.BlockSpec((NC, blk, LANES), lambda i: (0, i, 0))
    w_spec = pl.BlockSpec((d, d), lambda i: (0, 0))
    b_spec = pl.BlockSpec((1, d), lambda i: (0, 0))
    return pl.pallas_call(
        _out_body,
        grid=(grid,),
        in_specs=[acc_spec, den_spec, row_spec, b_spec, w_spec, b_spec],
        out_specs=row_spec,
        out_shape=jax.ShapeDtypeStruct((n, d), jnp.float32),
    )(accs, dens, h, bias2, whp, bhp2)


def kernel(x, adj, Wp, bp, Wl, Wr, att, bias, Wh, bh):
    n, d = x.shape
    e = adj.shape[1]
    nw = NC * NS
    npad = ((n + NS * CHUNK - 1) // (NS * CHUNK)) * (NS * CHUNK)
    quantum = nw * CHUNK * IDEPTH
    epad = ((e + quantum - 1) // quantum) * quantum
    src_flat = jnp.concatenate(
        [adj[0], jnp.zeros((epad - e,), jnp.int32)])
    dst_flat = jnp.concatenate(
        [adj[1], jnp.full((epad - e,), npad - 1, jnp.int32)])
    src3d = src_flat.reshape(nw, epad // (nw * CHUNK), CHUNK)
    dst3d = dst_flat.reshape(nw, epad // (nw * CHUNK), CHUNK)
    attb = att.reshape(d // LANES, LANES)

    h, hl, hr = _proj(x, Wp, bp.reshape(1, d), Wl, Wr)
    accs, dens = _sc_edges(hl, hr, src3d, dst3d, attb, npad)

    whp = jnp.pad(Wh, ((0, 0), (0, d - Wh.shape[1])))
    bhp2 = jnp.pad(bh, (0, d - bh.shape[0])).reshape(1, d)
    out = _combine(accs, dens, h, bias.reshape(1, d), whp, bhp2)
    return out[:, :1]


# speedup vs baseline: 14.0388x; 1.0006x over previous
import functools

import jax
import jax.numpy as jnp
from jax import lax
from jax.experimental import pallas as pl
from jax.experimental.pallas import tpu as pltpu
from jax.experimental.pallas import tpu_sc as plsc

NEG_SLOPE = 0.2
NC = 2
NS = 16
LANES = 16
CHUNK = 40
DEPTH = 3
IDEPTH = 6
EDGE_UNROLL = 8


def _proj_body(x_ref, wp_ref, bp_ref, wl_ref, wr_ref, h_ref, hl_ref, hr_ref):
    h = jnp.dot(x_ref[...], wp_ref[...],
                preferred_element_type=jnp.float32) + bp_ref[...]
    h_ref[...] = h
    hl_ref[...] = jnp.dot(h, wl_ref[...], preferred_element_type=jnp.float32)
    hr_ref[...] = jnp.dot(h, wr_ref[...], preferred_element_type=jnp.float32)


def _proj(x, wp, bp2, wl, wr):
    n, d = x.shape
    blk = 1000
    grid = n // blk
    row_spec = pl.BlockSpec((blk, d), lambda i: (i, 0))
    w_spec = pl.BlockSpec((d, d), lambda i: (0, 0))
    b_spec = pl.BlockSpec((1, d), lambda i: (0, 0))
    out_sd = jax.ShapeDtypeStruct((n, d), jnp.float32)
    return pl.pallas_call(
        _proj_body,
        grid=(grid,),
        in_specs=[row_spec, w_spec, b_spec, w_spec, w_spec],
        out_specs=[row_spec, row_spec, row_spec],
        out_shape=[out_sd, out_sd, out_sd],
    )(x, wp, bp2, wl, wr)


def _sc_edge_body(hl_hbm, hr_hbm, src_hbm, dst_hbm, attb_hbm,
                  acc_out, den_out,
                  src_c, dst_c, hlr, hrr, eev, attb_v,
                  acc_sh, den_sh, sin, sout, sidx):
    rows_per_tile = acc_sh.shape[0] // NS
    n_chunks = src_hbm.shape[1]

    c = lax.axis_index("c")
    s = lax.axis_index("s")
    wid = c * NS + s

    pltpu.sync_copy(attb_hbm, attb_v)

    zeros = jnp.zeros((LANES,), jnp.float32)

    def _z_rows(i, _):
        for g in range(8):
            hlr[0][i, pl.ds(g * LANES, LANES)] = zeros
        return 0
    lax.fori_loop(0, CHUNK, _z_rows, 0)

    for b in range(DEPTH):
        def _z_ee(i, _):
            eev[b][i] = zeros
            return 0
        lax.fori_loop(0, CHUNK, _z_ee, 0)

    for j in range(rows_per_tile // CHUNK):
        base = s * rows_per_tile + j * CHUNK
        pltpu.sync_copy(hlr[0], acc_sh.at[pl.ds(base, CHUNK)])
        pltpu.sync_copy(eev[0], den_sh.at[pl.ds(base, CHUNK)])
    plsc.subcore_barrier()

    av = [attb_v[g] for g in range(8)]
    zcol = jnp.zeros((LANES,), jnp.int32)

    def _issue_idx(cc, q):
        pltpu.async_copy(src_hbm.at[wid, cc], src_c[q], sidx[q])
        pltpu.async_copy(dst_hbm.at[wid, cc], dst_c[q], sidx[q])

    def _drain_idx(q):
        pltpu.make_async_copy(src_hbm.at[wid, 0], src_c[q], sidx[q]).wait()
        pltpu.make_async_copy(src_hbm.at[wid, 0], dst_c[q], sidx[q]).wait()

    def _issue_gather(q, b):
        pltpu.async_copy(hl_hbm.at[src_c[q]], hlr[b], sin[b])
        pltpu.async_copy(hr_hbm.at[dst_c[q]], hrr[b], sin[b])

    def _drain_gather(b):
        pltpu.make_async_copy(hl_hbm.at[pl.ds(0, CHUNK)], hlr[b],
                              sin[b]).wait()
        pltpu.make_async_copy(hl_hbm.at[pl.ds(0, CHUNK)], hrr[b],
                              sin[b]).wait()

    def _issue_scatter(q, b):
        pltpu.async_copy(hlr[b], acc_sh.at[dst_c[q]], sout[b], add=True)
        pltpu.async_copy(eev[b], den_sh.at[dst_c[q]], sout[b], add=True)

    def _drain_scatter(q):
        pltpu.make_async_copy(hl_hbm.at[pl.ds(0, CHUNK)], hlr[q],
                              sout[q]).wait()
        pltpu.make_async_copy(den_out.at[0, pl.ds(0, CHUNK)], eev[q],
                              sout[q]).wait()

    def _compute(q):
        def _edge(e4, _):
            for k in range(EDGE_UNROLL):
                e = e4 * EDGE_UNROLL + k
                acc = zeros
                for g in range(8):
                    a = hlr[q][e, pl.ds(g * LANES, LANES)]
                    b = hrr[q][e, pl.ds(g * LANES, LANES)]
                    z = a + b
                    acc = acc + jnp.maximum(z, NEG_SLOPE * z) * av[g]
                ee = jnp.exp(jnp.full((LANES,), jnp.sum(acc), jnp.float32))
                for g in range(8):
                    hlr[q][e, pl.ds(g * LANES, LANES)] = (
                        hlr[q][e, pl.ds(g * LANES, LANES)] * ee)
                plsc.store_scatter(
                    eev[q], [jnp.full((LANES,), e, jnp.int32), zcol], ee)
            return 0

        lax.fori_loop(0, CHUNK // EDGE_UNROLL, _edge, 0)

    _issue_idx(0, 0)
    _issue_idx(1, 1)
    _drain_idx(0)
    _issue_gather(0, 0)

    def _round(i, _):
        for b6 in range(IDEPTH):
            cc = i * IDEPTH + b6
            rb = b6 % DEPTH
            nrb = (rb + 1) % DEPTH

            @pl.when(cc >= 2)
            def _():
                _drain_scatter(nrb)

            @pl.when(cc + 1 < n_chunks)
            def _():
                _drain_idx((b6 + 1) % IDEPTH)
                _issue_gather((b6 + 1) % IDEPTH, nrb)

            @pl.when(cc + 2 < n_chunks)
            def _():
                _issue_idx(cc + 2, (b6 + 2) % IDEPTH)

            _drain_gather(rb)
            _compute(rb)
            _issue_scatter(b6, rb)
        return 0

    lax.fori_loop(0, n_chunks // IDEPTH, _round, 0)
    _drain_scatter((n_chunks - 2) % DEPTH)
    _drain_scatter((n_chunks - 1) % DEPTH)
    plsc.subcore_barrier()

    for j in range(rows_per_tile // CHUNK):
        base = s * rows_per_tile + j * CHUNK
        pltpu.sync_copy(acc_sh.at[pl.ds(base, CHUNK)], hlr[0])
        pltpu.sync_copy(hlr[0], acc_out.at[c, pl.ds(base, CHUNK)])
        pltpu.sync_copy(den_sh.at[pl.ds(base, CHUNK)], eev[0])
        pltpu.sync_copy(eev[0], den_out.at[c, pl.ds(base, CHUNK)])


def _sc_edges(hl, hr, src3d, dst3d, attb, npad):
    n, d = hl.shape
    rows_per_tile = npad // NS
    mesh = plsc.VectorSubcoreMesh(core_axis_name="c", subcore_axis_name="s",
                                  num_cores=NC, num_subcores=NS)
    f = pl.kernel(
        _sc_edge_body,
        out_type=[jax.ShapeDtypeStruct((NC, npad, d), jnp.float32),
                  jax.ShapeDtypeStruct((NC, npad, LANES), jnp.float32)],
        mesh=mesh,
        compiler_params=pltpu.CompilerParams(
            needs_layout_passes=False, use_tc_tiling_on_sc=False),
        scratch_types=[
            [pltpu.VMEM((CHUNK,), jnp.int32)] * IDEPTH,
            [pltpu.VMEM((CHUNK,), jnp.int32)] * IDEPTH,
            [pltpu.VMEM((CHUNK, d), jnp.float32)] * DEPTH,
            [pltpu.VMEM((CHUNK, d), jnp.float32)] * DEPTH,
            [pltpu.VMEM((CHUNK, LANES), jnp.float32)] * DEPTH,
            pltpu.VMEM((d // LANES, LANES), jnp.float32),
            pltpu.VMEM_SHARED((npad, d), jnp.float32),
            pltpu.VMEM_SHARED((npad, LANES), jnp.float32),
            [pltpu.SemaphoreType.DMA] * DEPTH,
            [pltpu.SemaphoreType.DMA] * DEPTH,
            [pltpu.SemaphoreType.DMA] * IDEPTH,
        ],
    )
    return f(hl, hr, src3d, dst3d, attb)


def _out_body(a_ref, d_ref, h_ref, bias_ref, wh_ref, bh_ref, o_ref):
    acc = a_ref[0] + a_ref[1]
    den = d_ref[0] + d_ref[1]
    denc = den[:, 0:1]
    o = acc / (denc + 1e-16) + h_ref[...] + bias_ref[...]
    o_ref[...] = jnp.dot(o, wh_ref[...],
                         preferred_element_type=jnp.float32) + bh_ref[...]


def _combine(accs, dens, h, bias2, whp, bhp2):
    n, d = h.shape
    blk = 1000
    grid = n // blk
    row_spec = pl.BlockSpec((blk, d), lambda i: (i, 0))
    acc_spec = pl.BlockSpec((NC, blk, d), lambda i: (0, i, 0))
    den_spec = pl.BlockSpec((NC, blk, LANES), lambda i: (0, i, 0))
    w_spec = pl.BlockSpec((d, d), lambda i: (0, 0))
    b_spec = pl.BlockSpec((1, d), lambda i: (0, 0))
    return pl.pallas_call(
        _out_body,
        grid=(grid,),
        in_specs=[acc_spec, den_spec, row_spec, b_spec, w_spec, b_spec],
        out_specs=row_spec,
        out_shape=jax.ShapeDtypeStruct((n, d), jnp.float32),
    )(accs, dens, h, bias2, whp, bhp2)


def kernel(x, adj, Wp, bp, Wl, Wr, att, bias, Wh, bh):
    n, d = x.shape
    e = adj.shape[1]
    nw = NC * NS
    npad = ((n + NS * CHUNK - 1) // (NS * CHUNK)) * (NS * CHUNK)
    quantum = nw * CHUNK * IDEPTH
    epad = ((e + quantum - 1) // quantum) * quantum
    src_flat = jnp.concatenate(
        [adj[0], jnp.zeros((epad - e,), jnp.int32)])
    dst_flat = jnp.concatenate(
        [adj[1], jnp.full((epad - e,), npad - 1, jnp.int32)])
    src3d = src_flat.reshape(nw, epad // (nw * CHUNK), CHUNK)
    dst3d = dst_flat.reshape(nw, epad // (nw * CHUNK), CHUNK)
    attb = att.reshape(d // LANES, LANES)

    h, hl, hr = _proj(x, Wp, bp.reshape(1, d), Wl, Wr)
    accs, dens = _sc_edges(hl, hr, src3d, dst3d, attb, npad)

    whp = jnp.pad(Wh, ((0, 0), (0, d - Wh.shape[1])))
    bhp2 = jnp.pad(bh, (0, d - bh.shape[0])).reshape(1, d)
    out = _combine(accs, dens, h, bias.reshape(1, d), whp, bhp2)
    return out[:, :1]


# speedup vs baseline: 14.3900x; 1.0250x over previous
import functools

import jax
import jax.numpy as jnp
from jax import lax
from jax.experimental import pallas as pl
from jax.experimental.pallas import tpu as pltpu
from jax.experimental.pallas import tpu_sc as plsc

NEG_SLOPE = 0.2
NC = 2
NS = 16
LANES = 16
CHUNK = 40
DEPTH = 3
IDEPTH = 6
EDGE_UNROLL = 8


def _proj_body(x_ref, wp_ref, bp_ref, wl_ref, wr_ref, h_ref, hl_ref, hr_ref):
    h = jnp.dot(x_ref[...], wp_ref[...],
                preferred_element_type=jnp.float32) + bp_ref[...]
    h_ref[...] = h
    hl_ref[...] = jnp.dot(h, wl_ref[...], preferred_element_type=jnp.float32)
    hr_ref[...] = jnp.dot(h, wr_ref[...], preferred_element_type=jnp.float32)


def _proj(x, wp, bp2, wl, wr):
    n, d = x.shape
    blk = 1000
    grid = n // blk
    row_spec = pl.BlockSpec((blk, d), lambda i: (i, 0))
    w_spec = pl.BlockSpec((d, d), lambda i: (0, 0))
    b_spec = pl.BlockSpec((1, d), lambda i: (0, 0))
    out_sd = jax.ShapeDtypeStruct((n, d), jnp.float32)
    return pl.pallas_call(
        _proj_body,
        grid=(grid,),
        in_specs=[row_spec, w_spec, b_spec, w_spec, w_spec],
        out_specs=[row_spec, row_spec, row_spec],
        out_shape=[out_sd, out_sd, out_sd],
    )(x, wp, bp2, wl, wr)


def _sc_edge_body(hl_hbm, hr_hbm, src_hbm, dst_hbm, attb_hbm,
                  acc_out, den_out,
                  src_c, dst_c, hlr, hrr, scr, eev, attb_v,
                  acc_sh, den_sh, sin, sout, sidx):
    rows_per_tile = acc_sh.shape[0] // NS
    n_chunks = src_hbm.shape[1]

    c = lax.axis_index("c")
    s = lax.axis_index("s")
    wid = c * NS + s

    pltpu.sync_copy(attb_hbm, attb_v)

    zeros = jnp.zeros((LANES,), jnp.float32)

    def _z_rows(i, _):
        for g in range(8):
            scr[0][i, pl.ds(g * LANES, LANES)] = zeros
        return 0
    lax.fori_loop(0, CHUNK, _z_rows, 0)

    for b in range(DEPTH):
        def _z_ee(i, _):
            eev[b][i] = zeros
            return 0
        lax.fori_loop(0, CHUNK, _z_ee, 0)

    for j in range(rows_per_tile // CHUNK):
        base = s * rows_per_tile + j * CHUNK
        pltpu.sync_copy(scr[0], acc_sh.at[pl.ds(base, CHUNK)])
        pltpu.sync_copy(eev[0], den_sh.at[pl.ds(base, CHUNK)])
    plsc.subcore_barrier()

    av = [attb_v[g] for g in range(8)]
    zcol = jnp.zeros((LANES,), jnp.int32)

    def _issue_idx(cc, q):
        pltpu.async_copy(src_hbm.at[wid, cc], src_c[q], sidx[q])
        pltpu.async_copy(dst_hbm.at[wid, cc], dst_c[q], sidx[q])

    def _drain_idx(q):
        pltpu.make_async_copy(src_hbm.at[wid, 0], src_c[q], sidx[q]).wait()
        pltpu.make_async_copy(src_hbm.at[wid, 0], dst_c[q], sidx[q]).wait()

    def _issue_gather(q, b):
        pltpu.async_copy(hl_hbm.at[src_c[q]], hlr[b], sin[b])
        pltpu.async_copy(hr_hbm.at[dst_c[q]], hrr[b], sin[b])

    def _drain_gather(b):
        pltpu.make_async_copy(hl_hbm.at[pl.ds(0, CHUNK)], hlr[b],
                              sin[b]).wait()
        pltpu.make_async_copy(hl_hbm.at[pl.ds(0, CHUNK)], hrr[b],
                              sin[b]).wait()

    def _issue_scatter(q, b):
        pltpu.async_copy(scr[b], acc_sh.at[dst_c[q]], sout[b], add=True)
        pltpu.async_copy(eev[b], den_sh.at[dst_c[q]], sout[b], add=True)

    def _drain_scatter(q):
        pltpu.make_async_copy(acc_out.at[0, pl.ds(0, CHUNK)], scr[q],
                              sout[q]).wait()
        pltpu.make_async_copy(den_out.at[0, pl.ds(0, CHUNK)], eev[q],
                              sout[q]).wait()

    iota16 = lax.iota(jnp.int32, LANES)
    colidx = [32 * g + 2 * iota16 + p for g in range(8 // 2)
              for p in range(2)]

    def _compute(q):
        def _edge(e4, _):
            for k in range(EDGE_UNROLL):
                e = e4 * EDGE_UNROLL + k
                acc = zeros
                hkeep = []
                for g in range(8 // 2):
                    hv = hlr[q][e, pl.ds(g * 2 * LANES, 2 * LANES)]
                    rv = hrr[q][e, pl.ds(g * 2 * LANES, 2 * LANES)]
                    ha, hb = plsc.unpack(
                        hv, format=plsc.PackFormat.INTERLEAVED,
                        preferred_element_type=jnp.float32)
                    ra, rb = plsc.unpack(
                        rv, format=plsc.PackFormat.INTERLEAVED,
                        preferred_element_type=jnp.float32)
                    hkeep += [ha, hb]
                    za = ha + ra
                    zb = hb + rb
                    la = jnp.maximum(za, NEG_SLOPE * za)
                    lb = jnp.maximum(zb, NEG_SLOPE * zb)
                    acc = acc + la * av[2 * g] + lb * av[2 * g + 1]
                ee = jnp.exp(jnp.full((LANES,), jnp.sum(acc), jnp.float32))
                erow = jnp.full((LANES,), e, jnp.int32)
                for g in range(8):
                    plsc.store_scatter(scr[q], [erow, colidx[g]],
                                       hkeep[g] * ee)
                plsc.store_scatter(eev[q], [erow, zcol], ee)
            return 0

        lax.fori_loop(0, CHUNK // EDGE_UNROLL, _edge, 0)

    _issue_idx(0, 0)
    _issue_idx(1, 1)
    _drain_idx(0)
    _issue_gather(0, 0)

    def _round(i, _):
        for b6 in range(IDEPTH):
            cc = i * IDEPTH + b6
            rb = b6 % DEPTH
            nrb = (rb + 1) % DEPTH

            @pl.when(cc >= 2)
            def _():
                _drain_scatter(nrb)

            @pl.when(cc + 1 < n_chunks)
            def _():
                _drain_idx((b6 + 1) % IDEPTH)
                _issue_gather((b6 + 1) % IDEPTH, nrb)

            @pl.when(cc + 2 < n_chunks)
            def _():
                _issue_idx(cc + 2, (b6 + 2) % IDEPTH)

            _drain_gather(rb)
            _compute(rb)
            _issue_scatter(b6, rb)
        return 0

    lax.fori_loop(0, n_chunks // IDEPTH, _round, 0)
    _drain_scatter((n_chunks - 2) % DEPTH)
    _drain_scatter((n_chunks - 1) % DEPTH)
    plsc.subcore_barrier()

    for j in range(rows_per_tile // CHUNK):
        base = s * rows_per_tile + j * CHUNK
        pltpu.sync_copy(acc_sh.at[pl.ds(base, CHUNK)], scr[0])
        pltpu.sync_copy(scr[0], acc_out.at[c, pl.ds(base, CHUNK)])
        pltpu.sync_copy(den_sh.at[pl.ds(base, CHUNK)], eev[0])
        pltpu.sync_copy(eev[0], den_out.at[c, pl.ds(base, CHUNK)])


def _sc_edges(hl, hr, src3d, dst3d, attb, npad):
    n, d = hl.shape
    rows_per_tile = npad // NS
    mesh = plsc.VectorSubcoreMesh(core_axis_name="c", subcore_axis_name="s",
                                  num_cores=NC, num_subcores=NS)
    f = pl.kernel(
        _sc_edge_body,
        out_type=[jax.ShapeDtypeStruct((NC, npad, d), jnp.float32),
                  jax.ShapeDtypeStruct((NC, npad, LANES), jnp.float32)],
        mesh=mesh,
        compiler_params=pltpu.CompilerParams(
            needs_layout_passes=False, use_tc_tiling_on_sc=False),
        scratch_types=[
            [pltpu.VMEM((CHUNK,), jnp.int32)] * IDEPTH,
            [pltpu.VMEM((CHUNK,), jnp.int32)] * IDEPTH,
            [pltpu.VMEM((CHUNK, d), jnp.bfloat16)] * DEPTH,
            [pltpu.VMEM((CHUNK, d), jnp.bfloat16)] * DEPTH,
            [pltpu.VMEM((CHUNK, d), jnp.float32)] * DEPTH,
            [pltpu.VMEM((CHUNK, LANES), jnp.float32)] * DEPTH,
            pltpu.VMEM((d // LANES, LANES), jnp.float32),
            pltpu.VMEM_SHARED((npad, d), jnp.float32),
            pltpu.VMEM_SHARED((npad, LANES), jnp.float32),
            [pltpu.SemaphoreType.DMA] * DEPTH,
            [pltpu.SemaphoreType.DMA] * DEPTH,
            [pltpu.SemaphoreType.DMA] * IDEPTH,
        ],
    )
    return f(hl, hr, src3d, dst3d, attb)


def _out_body(a_ref, d_ref, h_ref, bias_ref, wh_ref, bh_ref, o_ref):
    acc = a_ref[0] + a_ref[1]
    den = d_ref[0] + d_ref[1]
    denc = den[:, 0:1]
    o = acc / (denc + 1e-16) + h_ref[...] + bias_ref[...]
    o_ref[...] = jnp.dot(o, wh_ref[...],
                         preferred_element_type=jnp.float32) + bh_ref[...]


def _combine(accs, dens, h, bias2, whp, bhp2):
    n, d = h.shape
    blk = 1000
    grid = n // blk
    row_spec = pl.BlockSpec((blk, d), lambda i: (i, 0))
    acc_spec = pl.BlockSpec((NC, blk, d), lambda i: (0, i, 0))
    den_spec = pl.BlockSpec((NC, blk, LANES), lambda i: (0, i, 0))
    w_spec = pl.BlockSpec((d, d), lambda i: (0, 0))
    b_spec = pl.BlockSpec((1, d), lambda i: (0, 0))
    return pl.pallas_call(
        _out_body,
        grid=(grid,),
        in_specs=[acc_spec, den_spec, row_spec, b_spec, w_spec, b_spec],
        out_specs=row_spec,
        out_shape=jax.ShapeDtypeStruct((n, d), jnp.float32),
    )(accs, dens, h, bias2, whp, bhp2)


def kernel(x, adj, Wp, bp, Wl, Wr, att, bias, Wh, bh):
    n, d = x.shape
    e = adj.shape[1]
    nw = NC * NS
    npad = ((n + NS * CHUNK - 1) // (NS * CHUNK)) * (NS * CHUNK)
    quantum = nw * CHUNK * IDEPTH
    epad = ((e + quantum - 1) // quantum) * quantum
    src_flat = jnp.concatenate(
        [adj[0], jnp.zeros((epad - e,), jnp.int32)])
    dst_flat = jnp.concatenate(
        [adj[1], jnp.full((epad - e,), npad - 1, jnp.int32)])
    src3d = src_flat.reshape(nw, epad // (nw * CHUNK), CHUNK)
    dst3d = dst_flat.reshape(nw, epad // (nw * CHUNK), CHUNK)
    attb = att.reshape(d // 32, LANES, 2).transpose(0, 2, 1).reshape(
        d // LANES, LANES)

    h, hl, hr = _proj(x, Wp, bp.reshape(1, d), Wl, Wr)
    accs, dens = _sc_edges(hl.astype(jnp.bfloat16), hr.astype(jnp.bfloat16),
                           src3d, dst3d, attb, npad)

    whp = jnp.pad(Wh, ((0, 0), (0, d - Wh.shape[1])))
    bhp2 = jnp.pad(bh, (0, d - bh.shape[0])).reshape(1, d)
    out = _combine(accs, dens, h, bias.reshape(1, d), whp, bhp2)
    return out[:, :1]
